# baseline (speedup 1.0000x reference)
import functools

import jax
import jax.numpy as jnp
from jax import lax
from jax.experimental import pallas as pl
from jax.experimental.pallas import tpu as pltpu
from jax.experimental.pallas import tpu_sc as plsc

_N_NODES = 10000
_D = 128
_N_EDGES = 320000

_NC = 2
_NS = 16
_NW = _NC * _NS
_LANES = 16
_E_PER = 10112
_LAST_LO = _N_EDGES - _E_PER
_Q_OFF = 10240


def _tc_body(x_ref, wt_ref, bias_ref, pq_ref):
    bias = jnp.concatenate(
        [bias_ref[...], jnp.zeros_like(bias_ref[...])], axis=0
    )
    pq = (
        lax.dot_general(
            wt_ref[...], x_ref[...],
            dimension_numbers=(((1,), (1,)), ((), ())),
            preferred_element_type=jnp.float32,
        )
        + bias
    )
    pq_ref[pl.ds(0, _N_NODES)] = pq[0:1, :].reshape(_N_NODES)
    pq_ref[pl.ds(_Q_OFF, _N_NODES)] = pq[1:2, :].reshape(_N_NODES)


def _tc_prep(x, W, b):
    wt = W.reshape(2, _D)
    bias = b.reshape(1, 1)
    return pl.pallas_call(
        _tc_body,
        out_shape=jax.ShapeDtypeStruct((_Q_OFF + _N_NODES,), jnp.float32),
    )(x, wt, bias)


def _sc_body(
    table_hbm, edges_hbm, out_hbm, tab_v, win_v, out_v, tab_sh, sem, sem2, sem3
):
    wid = lax.axis_index("s") * _NC + lax.axis_index("c")
    base = jnp.minimum(wid * _E_PER, _LAST_LO)
    cut = 5120
    c1 = pltpu.async_copy(
        edges_hbm.at[:, pl.ds(base, cut)], win_v.at[:, pl.ds(0, cut)], sem2
    )
    @pl.when(lax.axis_index("s") == 0)
    def _():
        pltpu.sync_copy(table_hbm, tab_sh)

    plsc.subcore_barrier()
    c0 = pltpu.async_copy(tab_sh, tab_v, sem)
    c2 = pltpu.async_copy(
        edges_hbm.at[:, pl.ds(base + cut, _E_PER - cut)],
        win_v.at[:, pl.ds(cut, _E_PER - cut)],
        sem3,
    )
    c1.wait()
    c0.wait()

    def run(lo, hi):
        @plsc.parallel_loop(lo, hi, step=_LANES, unroll=16)
        def _(i):
            sl = pl.ds(i, _LANES)
            si = win_v[0, sl]
            di = win_v[1, sl]
            pv = plsc.load_gather(tab_v, [si])
            qv = plsc.load_gather(tab_v, [di + _Q_OFF])
            out_v[sl] = pv + qv

    run(0, cut)
    c3 = pltpu.async_copy(
        out_v.at[pl.ds(0, cut)], out_hbm.at[0, pl.ds(base, cut)], sem2
    )
    c2.wait()
    run(cut, _E_PER)
    c4 = pltpu.async_copy(
        out_v.at[pl.ds(cut, _E_PER - cut)],
        out_hbm.at[0, pl.ds(base + cut, _E_PER - cut)],
        sem3,
    )
    c3.wait()
    c4.wait()


@functools.cache
def _sc_gather():
    return pl.kernel(
        _sc_body,
        mesh=plsc.VectorSubcoreMesh(core_axis_name="c", subcore_axis_name="s"),
        compiler_params=pltpu.CompilerParams(needs_layout_passes=False),
        out_type=jax.ShapeDtypeStruct((1, _N_EDGES), jnp.float32),
        scratch_types=[
            pltpu.VMEM((_Q_OFF + _N_NODES,), jnp.float32),
            pltpu.VMEM((2, _E_PER), jnp.int32),
            pltpu.VMEM((_E_PER,), jnp.float32),
            pltpu.VMEM_SHARED((_Q_OFF + _N_NODES,), jnp.float32),
            pltpu.SemaphoreType.DMA,
            pltpu.SemaphoreType.DMA,
            pltpu.SemaphoreType.DMA,
        ],
    )


def kernel(x, edge_label_index, W, b):
    table = _tc_prep(x, W, b)
    out = _sc_gather()(table, edge_label_index.astype(jnp.int32))
    return out.reshape(_N_EDGES, 1)


# speedup vs baseline: 1.0123x; 1.0123x over previous
import functools

import jax
import jax.numpy as jnp
from jax import lax
from jax.experimental import pallas as pl
from jax.experimental.pallas import tpu as pltpu
from jax.experimental.pallas import tpu_sc as plsc

_N_NODES = 10000
_D = 128
_N_EDGES = 320000

_NC = 2
_NS = 16
_NW = _NC * _NS
_LANES = 16
_E_PER = 10112
_LAST_LO = _N_EDGES - _E_PER
_Q_OFF = 10240


def _tc_body(x_ref, wt_ref, bias_ref, pq_ref):
    bias = jnp.concatenate(
        [bias_ref[...], jnp.zeros_like(bias_ref[...])], axis=0
    )
    pq = (
        lax.dot_general(
            wt_ref[...], x_ref[...],
            dimension_numbers=(((1,), (1,)), ((), ())),
            preferred_element_type=jnp.float32,
        )
        + bias
    )
    pq_ref[pl.ds(0, _N_NODES)] = pq[0:1, :].reshape(_N_NODES)
    pq_ref[pl.ds(_Q_OFF, _N_NODES)] = pq[1:2, :].reshape(_N_NODES)


def _tc_prep(x, W, b):
    wt = W.reshape(2, _D)
    bias = b.reshape(1, 1)
    return pl.pallas_call(
        _tc_body,
        out_shape=jax.ShapeDtypeStruct((_Q_OFF + _N_NODES,), jnp.float32),
    )(x, wt, bias)


def _sc_body(
    table_hbm, edges_hbm, out_hbm, tab_v, win_v, out_v, tab_sh, sem, sem2, sem3
):
    wid = lax.axis_index("s") * _NC + lax.axis_index("c")
    base = jnp.minimum(wid * _E_PER, _LAST_LO)
    cut = 5120
    c1 = pltpu.async_copy(
        edges_hbm.at[:, pl.ds(base, cut)], win_v.at[:, pl.ds(0, cut)], sem2
    )
    @pl.when(lax.axis_index("s") == 0)
    def _():
        pltpu.sync_copy(table_hbm, tab_sh)

    plsc.subcore_barrier()
    c0 = pltpu.async_copy(tab_sh, tab_v, sem)
    c2 = pltpu.async_copy(
        edges_hbm.at[:, pl.ds(base + cut, _E_PER - cut)],
        win_v.at[:, pl.ds(cut, _E_PER - cut)],
        sem3,
    )
    c1.wait()
    c0.wait()

    def run(lo, hi):
        @plsc.parallel_loop(lo, hi, step=_LANES, unroll=8)
        def _(i):
            sl = pl.ds(i, _LANES)
            si = win_v[0, sl]
            di = win_v[1, sl]
            pv = plsc.load_gather(tab_v, [si])
            qv = plsc.load_gather(tab_v, [di + _Q_OFF])
            out_v[sl] = pv + qv

    run(0, cut)
    c3 = pltpu.async_copy(
        out_v.at[pl.ds(0, cut)], out_hbm.at[0, pl.ds(base, cut)], sem2
    )
    c2.wait()
    run(cut, _E_PER)
    c4 = pltpu.async_copy(
        out_v.at[pl.ds(cut, _E_PER - cut)],
        out_hbm.at[0, pl.ds(base + cut, _E_PER - cut)],
        sem3,
    )
    c3.wait()
    c4.wait()


@functools.cache
def _sc_gather():
    return pl.kernel(
        _sc_body,
        mesh=plsc.VectorSubcoreMesh(core_axis_name="c", subcore_axis_name="s"),
        compiler_params=pltpu.CompilerParams(needs_layout_passes=False),
        out_type=jax.ShapeDtypeStruct((1, _N_EDGES), jnp.float32),
        scratch_types=[
            pltpu.VMEM((_Q_OFF + _N_NODES,), jnp.float32),
            pltpu.VMEM((2, _E_PER), jnp.int32),
            pltpu.VMEM((_E_PER,), jnp.float32),
            pltpu.VMEM_SHARED((_Q_OFF + _N_NODES,), jnp.float32),
            pltpu.SemaphoreType.DMA,
            pltpu.SemaphoreType.DMA,
            pltpu.SemaphoreType.DMA,
        ],
    )


def kernel(x, edge_label_index, W, b):
    table = _tc_prep(x, W, b)
    out = _sc_gather()(table, edge_label_index.astype(jnp.int32))
    return out.reshape(_N_EDGES, 1)
